# Initial kernel scaffold; baseline (speedup 1.0000x reference)
import jax
import jax.numpy as jnp
from jax.experimental import pallas as pl


def kernel(inputs):
    raise NotImplementedError("write your pallas kernel here")


# speedup vs baseline: 49.7989x; 49.7989x over previous
import jax
import jax.numpy as jnp
from jax.experimental import pallas as pl

_K = 128
_LOGK = 7
_HB = 1024
_CB = 128


def _cmpx(x, d, level, invert):
    r = x.shape[0]
    iota = jax.lax.broadcasted_iota(jnp.int32, (r, 1), 0)
    bit = (iota & d) != 0
    up = jnp.roll(x, -d, axis=0)
    down = jnp.roll(x, d, axis=0)
    partner = jnp.where(bit, down, up)
    hi = jnp.maximum(x, partner)
    lo = jnp.minimum(x, partner)
    desc = ((iota >> level) & 1) == (1 if invert else 0)
    keep_hi = jnp.logical_xor(desc, bit)
    return jnp.where(keep_hi, hi, lo)


def _top_k_ascending(x):
    r, cb = x.shape
    for level in range(1, _LOGK + 1):
        for t in range(level):
            x = _cmpx(x, 1 << (level - 1 - t), level, invert=True)
    while r > _K:
        paired_max = jnp.maximum(x, jnp.roll(x, -_K, axis=0))
        x = paired_max.reshape(r // (2 * _K), 2, _K, cb)[:, 0].reshape(r // 2, cb)
        r //= 2
        for t in range(_LOGK):
            x = _cmpx(x, 1 << (_LOGK - 1 - t), _LOGK, invert=True)
    return x


def _kmax_kernel(x_ref, o_ref):
    hstep = pl.program_id(2)

    @pl.when(hstep == 0)
    def _init():
        o_ref[...] = jnp.full_like(o_ref, -jnp.inf)

    cand = _top_k_ascending(x_ref[...])
    acc = o_ref[...]
    m = jnp.maximum(acc, cand)
    for t in range(_LOGK):
        m = _cmpx(m, 1 << (_LOGK - 1 - t), _LOGK, invert=False)
    o_ref[...] = m


def kernel(inputs):
    b, h, w, c = inputs.shape
    out = pl.pallas_call(
        _kmax_kernel,
        grid=(b, c // _CB, h // _HB),
        in_specs=[pl.BlockSpec((None, _HB, None, _CB),
                               lambda i, j, s: (i, s, 0, j))],
        out_specs=pl.BlockSpec((None, _K, None, _CB),
                               lambda i, j, s: (i, 0, 0, j)),
        out_shape=jax.ShapeDtypeStruct((b, _K, w, c), inputs.dtype),
    )(inputs)
    return out


# speedup vs baseline: 53.1652x; 1.0676x over previous
import jax
import jax.numpy as jnp
from jax.experimental import pallas as pl

_K = 128
_LOGK = 7
_HB = 1024
_CB = 128


def _cmpx(x, d, level, invert):
    r, cb = x.shape
    if d >= 8:
        m = r // (2 * d)
        y = x.reshape(m, 2, d, cb)
        p0 = y[:, 0]
        p1 = y[:, 1]
        hi = jnp.maximum(p0, p1)
        lo = jnp.minimum(p0, p1)
        shift = level - (d.bit_length())
        giota = jax.lax.broadcasted_iota(jnp.int32, (m, 1, 1), 0)
        desc = ((giota >> shift) & 1) == (1 if invert else 0)
        out = jnp.concatenate(
            [jnp.where(desc, hi, lo)[:, None], jnp.where(desc, lo, hi)[:, None]],
            axis=1)
        return out.reshape(r, cb)
    iota = jax.lax.broadcasted_iota(jnp.int32, (r, 1), 0)
    bit = (iota & d) != 0
    up = jnp.roll(x, -d, axis=0)
    down = jnp.roll(x, d, axis=0)
    partner = jnp.where(bit, down, up)
    hi = jnp.maximum(x, partner)
    lo = jnp.minimum(x, partner)
    desc = ((iota >> level) & 1) == (1 if invert else 0)
    keep_hi = jnp.logical_xor(desc, bit)
    return jnp.where(keep_hi, hi, lo)


def _top_k_ascending(x):
    r, cb = x.shape
    for level in range(1, _LOGK + 1):
        for t in range(level):
            x = _cmpx(x, 1 << (level - 1 - t), level, invert=True)
    while r > _K:
        y = x.reshape(r // (2 * _K), 2, _K, cb)
        x = jnp.maximum(y[:, 0], y[:, 1]).reshape(r // 2, cb)
        r //= 2
        for t in range(_LOGK):
            x = _cmpx(x, 1 << (_LOGK - 1 - t), _LOGK, invert=True)
    return x


def _kmax_kernel(x_ref, o_ref):
    hstep = pl.program_id(2)

    @pl.when(hstep == 0)
    def _init():
        o_ref[...] = jnp.full_like(o_ref, -jnp.inf)

    cand = _top_k_ascending(x_ref[...])
    acc = o_ref[...]
    m = jnp.maximum(acc, cand)
    for t in range(_LOGK):
        m = _cmpx(m, 1 << (_LOGK - 1 - t), _LOGK, invert=False)
    o_ref[...] = m


def kernel(inputs):
    b, h, w, c = inputs.shape
    out = pl.pallas_call(
        _kmax_kernel,
        grid=(b, c // _CB, h // _HB),
        in_specs=[pl.BlockSpec((None, _HB, None, _CB),
                               lambda i, j, s: (i, s, 0, j))],
        out_specs=pl.BlockSpec((None, _K, None, _CB),
                               lambda i, j, s: (i, 0, 0, j)),
        out_shape=jax.ShapeDtypeStruct((b, _K, w, c), inputs.dtype),
    )(inputs)
    return out


# speedup vs baseline: 79.3401x; 1.4923x over previous
import numpy as np
import jax
import jax.numpy as jnp
from jax.experimental import pallas as pl

_K = 128
_LOGK = 7
_HB = 1024
_CB = 128


def _host_sign_columns():
    p = np.arange(_HB)

    def bit(b):
        return (p >> b) & 1

    cols = [
        bit(4),
        bit(4) ^ bit(5),
        bit(5) ^ bit(6),
        bit(6) ^ bit(7),
        bit(7) ^ bit(8),
        bit(8) ^ bit(9),
        bit(9) ^ bit(0),
        bit(0),
        bit(1),
        bit(2),
    ]
    sgn = 1.0 - 2.0 * np.stack(cols, axis=1).astype(np.float32)
    out = np.ones((_HB, 16), dtype=np.float32)
    out[:, : len(cols)] = sgn
    return jnp.asarray(out)


def _clean_stage(x, dp):
    r, cb = x.shape
    if dp >= 128:
        pieces = []
        for g in range(r // (2 * dp)):
            a = x[g * 2 * dp:g * 2 * dp + dp]
            b = x[g * 2 * dp + dp:(g + 1) * 2 * dp]
            pieces.append(jnp.maximum(a, b))
            pieces.append(jnp.minimum(a, b))
        return jnp.concatenate(pieces, axis=0)
    y = x.reshape(r // (2 * dp), 2, dp, cb)
    p0 = y[:, 0]
    p1 = y[:, 1]
    hi = jnp.maximum(p0, p1)
    lo = jnp.minimum(p0, p1)
    return jnp.concatenate([hi[:, None], lo[:, None]], axis=1).reshape(r, cb)


def _cmpx(x, d, level):
    r, cb = x.shape
    if d >= 8:
        m = r // (2 * d)
        y = x.reshape(m, 2, d, cb)
        p0 = y[:, 0:1]
        p1 = y[:, 1:2]
        return jnp.concatenate(
            [jnp.maximum(p0, p1), jnp.minimum(p0, p1)], axis=1).reshape(r, cb)
    iota = jax.lax.broadcasted_iota(jnp.int32, (r, 1), 0)
    bit = (iota & d) != 0
    up = jnp.roll(x, -d, axis=0)
    down = jnp.roll(x, d, axis=0)
    partner = jnp.where(bit, down, up)
    hi = jnp.maximum(x, partner)
    lo = jnp.minimum(x, partner)
    desc = ((iota >> level) & 1) == 0
    keep_hi = jnp.logical_xor(desc, bit)
    return jnp.where(keep_hi, hi, lo)


def _kmax_kernel(x_ref, s_ref, o_ref):
    hstep = pl.program_id(2)

    @pl.when(hstep == 0)
    def _init():
        o_ref[...] = jnp.full_like(o_ref, -jnp.inf)

    x = x_ref[...]
    cb = x.shape[1]

    def sgn(col):
        return s_ref[:, col:col + 1]

    x = x * sgn(0)
    for level in range(1, _LOGK + 1):
        for t in range(level):
            x = _clean_stage(x, 8 << (level - 1 - t))
        if level < _LOGK:
            x = x * sgn(level)

    iota = jax.lax.broadcasted_iota(jnp.int32, (_HB, 1), 0)
    for j in range(3):
        t = x * sgn(7 + j)
        dj = 1 << j
        bitb = (iota & dj) != 0
        partner = jnp.where(bitb, jnp.roll(t, dj, axis=0),
                            jnp.roll(t, -dj, axis=0))
        m = jnp.maximum(t, partner)
        if j < 2:
            x = m * sgn(8 + j)
        else:
            x = -m
        for u in range(_LOGK):
            x = _clean_stage(x, 8 << (_LOGK - 1 - u))

    cand = -x.reshape(_K, 8, cb)[:, 0, :]
    acc = o_ref[...]
    m = jnp.maximum(acc, cand)
    for u in range(_LOGK):
        m = _cmpx(m, 1 << (_LOGK - 1 - u), _LOGK)
    o_ref[...] = m


def kernel(inputs):
    b, h, w, c = inputs.shape
    signs = _host_sign_columns()
    out = pl.pallas_call(
        _kmax_kernel,
        grid=(b, c // _CB, h // _HB),
        in_specs=[
            pl.BlockSpec((None, _HB, None, _CB), lambda i, j, s: (i, s, 0, j)),
            pl.BlockSpec((_HB, 16), lambda i, j, s: (0, 0)),
        ],
        out_specs=pl.BlockSpec((None, _K, None, _CB),
                               lambda i, j, s: (i, 0, 0, j)),
        out_shape=jax.ShapeDtypeStruct((b, _K, w, c), inputs.dtype),
    )(inputs, signs)
    return out


# speedup vs baseline: 123.0616x; 1.5511x over previous
import jax
import jax.numpy as jnp
from jax.experimental import pallas as pl

_K = 128
_LOGK = 7
_HB = 1024
_CB = 128


def _rev8(x):
    r = x.shape[0]
    return jnp.concatenate([x[i:i + 8] for i in range(r - 8, -1, -8)], axis=0)


def _clean_stage(x, dp):
    r, cb = x.shape
    if dp >= 128:
        pieces = []
        for g in range(r // (2 * dp)):
            a = x[g * 2 * dp:g * 2 * dp + dp]
            b = x[g * 2 * dp + dp:(g + 1) * 2 * dp]
            pieces.append(jnp.maximum(a, b))
            pieces.append(jnp.minimum(a, b))
        return jnp.concatenate(pieces, axis=0)
    y = x.reshape(r // (2 * dp), 2, dp, cb)
    p0 = y[:, 0]
    p1 = y[:, 1]
    hi = jnp.maximum(p0, p1)
    lo = jnp.minimum(p0, p1)
    return jnp.concatenate([hi[:, None], lo[:, None]], axis=1).reshape(r, cb)


def _merge_level(x, dp):
    r, cb = x.shape
    pieces = []
    for g in range(r // (2 * dp)):
        a = x[g * 2 * dp:g * 2 * dp + dp]
        rb = _rev8(x[g * 2 * dp + dp:(g + 1) * 2 * dp])
        pieces.append(jnp.maximum(a, rb))
        pieces.append(jnp.minimum(a, rb))
    x = jnp.concatenate(pieces, axis=0)
    d = dp // 2
    while d >= 8:
        x = _clean_stage(x, d)
        d //= 2
    return x


def _cmpx(x, d, level):
    r, cb = x.shape
    if d >= 8:
        m = r // (2 * d)
        y = x.reshape(m, 2, d, cb)
        p0 = y[:, 0]
        p1 = y[:, 1]
        hi = jnp.maximum(p0, p1)
        lo = jnp.minimum(p0, p1)
        return jnp.concatenate([hi[:, None], lo[:, None]], axis=1).reshape(r, cb)
    iota = jax.lax.broadcasted_iota(jnp.int32, (r, 1), 0)
    bit = (iota & d) != 0
    up = jnp.roll(x, -d, axis=0)
    down = jnp.roll(x, d, axis=0)
    partner = jnp.where(bit, down, up)
    hi = jnp.maximum(x, partner)
    lo = jnp.minimum(x, partner)
    desc = ((iota >> level) & 1) == 0
    keep_hi = jnp.logical_xor(desc, bit)
    return jnp.where(keep_hi, hi, lo)


def _kmax_kernel(x_ref, o_ref):
    hstep = pl.program_id(2)

    @pl.when(hstep == 0)
    def _init():
        o_ref[...] = jnp.full_like(o_ref, -jnp.inf)

    x = x_ref[...]
    cb = x.shape[1]

    x = _clean_stage(x, 8)
    for level in range(2, _LOGK + 1):
        x = _merge_level(x, 8 << (level - 1))

    iota = jax.lax.broadcasted_iota(jnp.int32, (_HB, 1), 0)
    for j in range(3):
        dj = 1 << j
        bitb = (iota & dj) != 0
        partner = jnp.where(bitb, jnp.roll(x, dj, axis=0),
                            jnp.roll(x, -dj, axis=0))
        x = jnp.maximum(x, _rev8(partner))
        d = 8 << (_LOGK - 1)
        while d >= 8:
            x = _clean_stage(x, d)
            d //= 2

    cand = _rev8(x).reshape(_K, 8, cb)[:, 0, :]
    acc = o_ref[...]
    m = jnp.maximum(acc, cand)
    for u in range(_LOGK):
        m = _cmpx(m, 1 << (_LOGK - 1 - u), _LOGK)
    o_ref[...] = m


def kernel(inputs):
    b, h, w, c = inputs.shape
    out = pl.pallas_call(
        _kmax_kernel,
        grid=(b, c // _CB, h // _HB),
        in_specs=[
            pl.BlockSpec((None, _HB, None, _CB), lambda i, j, s: (i, s, 0, j)),
        ],
        out_specs=pl.BlockSpec((None, _K, None, _CB),
                               lambda i, j, s: (i, 0, 0, j)),
        out_shape=jax.ShapeDtypeStruct((b, _K, w, c), inputs.dtype),
    )(inputs)
    return out


# speedup vs baseline: 147.0095x; 1.1946x over previous
import jax
import jax.numpy as jnp
from jax.experimental import pallas as pl

_K = 128
_LOGK = 7
_HB = 2048
_CB = 128


def _rev8(x):
    r = x.shape[0]
    return jnp.concatenate([x[i:i + 8] for i in range(r - 8, -1, -8)], axis=0)


def _clean_stage(x, dp):
    r, cb = x.shape
    if dp >= 128:
        pieces = []
        for g in range(r // (2 * dp)):
            a = x[g * 2 * dp:g * 2 * dp + dp]
            b = x[g * 2 * dp + dp:(g + 1) * 2 * dp]
            pieces.append(jnp.maximum(a, b))
            pieces.append(jnp.minimum(a, b))
        return jnp.concatenate(pieces, axis=0)
    y = x.reshape(r // (2 * dp), 2, dp, cb)
    p0 = y[:, 0]
    p1 = y[:, 1]
    hi = jnp.maximum(p0, p1)
    lo = jnp.minimum(p0, p1)
    return jnp.concatenate([hi[:, None], lo[:, None]], axis=1).reshape(r, cb)


def _merge_level(x, dp):
    r, cb = x.shape
    pieces = []
    for g in range(r // (2 * dp)):
        a = x[g * 2 * dp:g * 2 * dp + dp]
        rb = _rev8(x[g * 2 * dp + dp:(g + 1) * 2 * dp])
        pieces.append(jnp.maximum(a, rb))
        pieces.append(jnp.minimum(a, rb))
    x = jnp.concatenate(pieces, axis=0)
    d = dp // 2
    while d >= 8:
        x = _clean_stage(x, d)
        d //= 2
    return x


def _cmpx(x, d, level):
    r, cb = x.shape
    if d >= 8:
        m = r // (2 * d)
        y = x.reshape(m, 2, d, cb)
        p0 = y[:, 0]
        p1 = y[:, 1]
        hi = jnp.maximum(p0, p1)
        lo = jnp.minimum(p0, p1)
        return jnp.concatenate([hi[:, None], lo[:, None]], axis=1).reshape(r, cb)
    iota = jax.lax.broadcasted_iota(jnp.int32, (r, 1), 0)
    bit = (iota & d) != 0
    up = jnp.roll(x, -d, axis=0)
    down = jnp.roll(x, d, axis=0)
    partner = jnp.where(bit, down, up)
    hi = jnp.maximum(x, partner)
    lo = jnp.minimum(x, partner)
    desc = ((iota >> level) & 1) == 0
    keep_hi = jnp.logical_xor(desc, bit)
    return jnp.where(keep_hi, hi, lo)


def _kmax_kernel(x_ref, o_ref):
    hstep = pl.program_id(2)

    @pl.when(hstep == 0)
    def _init():
        o_ref[...] = jnp.full_like(o_ref, -jnp.inf)

    x = x_ref[...]
    cb = x.shape[1]

    x = _clean_stage(x, 8)
    for level in range(2, _LOGK + 1):
        x = _merge_level(x, 8 << (level - 1))

    while x.shape[0] > 1024:
        half = x.shape[0] // 2
        x = jnp.maximum(x[:half], _rev8(x[half:]))
        d = 8 << (_LOGK - 1)
        while d >= 8:
            x = _clean_stage(x, d)
            d //= 2

    iota = jax.lax.broadcasted_iota(jnp.int32, (1024, 1), 0)
    for j in range(3):
        dj = 1 << j
        bitb = (iota & dj) != 0
        partner = jnp.where(bitb, jnp.roll(x, dj, axis=0),
                            jnp.roll(x, -dj, axis=0))
        x = jnp.maximum(x, _rev8(partner))
        d = 8 << (_LOGK - 1)
        while d >= 8:
            x = _clean_stage(x, d)
            d //= 2

    cand = _rev8(x).reshape(_K, 8, cb)[:, 0, :]
    acc = o_ref[...]
    m = jnp.maximum(acc, cand)
    for u in range(_LOGK):
        m = _cmpx(m, 1 << (_LOGK - 1 - u), _LOGK)
    o_ref[...] = m


def kernel(inputs):
    b, h, w, c = inputs.shape
    out = pl.pallas_call(
        _kmax_kernel,
        grid=(b, c // _CB, h // _HB),
        in_specs=[
            pl.BlockSpec((None, _HB, None, _CB), lambda i, j, s: (i, s, 0, j)),
        ],
        out_specs=pl.BlockSpec((None, _K, None, _CB),
                               lambda i, j, s: (i, 0, 0, j)),
        out_shape=jax.ShapeDtypeStruct((b, _K, w, c), inputs.dtype),
    )(inputs)
    return out


# speedup vs baseline: 162.5908x; 1.1060x over previous
import jax
import jax.numpy as jnp
from jax.experimental import pallas as pl

_K = 128
_LOGK = 7
_HB = 4096
_CB = 128


def _rev8(x):
    r = x.shape[0]
    return jnp.concatenate([x[i:i + 8] for i in range(r - 8, -1, -8)], axis=0)


def _clean_stage(x, dp):
    r, cb = x.shape
    if dp >= 128:
        pieces = []
        for g in range(r // (2 * dp)):
            a = x[g * 2 * dp:g * 2 * dp + dp]
            b = x[g * 2 * dp + dp:(g + 1) * 2 * dp]
            pieces.append(jnp.maximum(a, b))
            pieces.append(jnp.minimum(a, b))
        return jnp.concatenate(pieces, axis=0)
    y = x.reshape(r // (2 * dp), 2, dp, cb)
    p0 = y[:, 0]
    p1 = y[:, 1]
    hi = jnp.maximum(p0, p1)
    lo = jnp.minimum(p0, p1)
    return jnp.concatenate([hi[:, None], lo[:, None]], axis=1).reshape(r, cb)


def _merge_level(x, dp):
    r, cb = x.shape
    pieces = []
    for g in range(r // (2 * dp)):
        a = x[g * 2 * dp:g * 2 * dp + dp]
        rb = _rev8(x[g * 2 * dp + dp:(g + 1) * 2 * dp])
        pieces.append(jnp.maximum(a, rb))
        pieces.append(jnp.minimum(a, rb))
    x = jnp.concatenate(pieces, axis=0)
    d = dp // 2
    while d >= 8:
        x = _clean_stage(x, d)
        d //= 2
    return x


def _cmpx(x, d, level):
    r, cb = x.shape
    if d >= 8:
        m = r // (2 * d)
        y = x.reshape(m, 2, d, cb)
        p0 = y[:, 0]
        p1 = y[:, 1]
        hi = jnp.maximum(p0, p1)
        lo = jnp.minimum(p0, p1)
        return jnp.concatenate([hi[:, None], lo[:, None]], axis=1).reshape(r, cb)
    iota = jax.lax.broadcasted_iota(jnp.int32, (r, 1), 0)
    bit = (iota & d) != 0
    up = jnp.roll(x, -d, axis=0)
    down = jnp.roll(x, d, axis=0)
    partner = jnp.where(bit, down, up)
    hi = jnp.maximum(x, partner)
    lo = jnp.minimum(x, partner)
    desc = ((iota >> level) & 1) == 0
    keep_hi = jnp.logical_xor(desc, bit)
    return jnp.where(keep_hi, hi, lo)


def _kmax_kernel(x_ref, o_ref):
    hstep = pl.program_id(2)

    @pl.when(hstep == 0)
    def _init():
        o_ref[...] = jnp.full_like(o_ref, -jnp.inf)

    x = x_ref[...]
    cb = x.shape[1]

    x = _clean_stage(x, 8)
    for level in range(2, _LOGK + 1):
        x = _merge_level(x, 8 << (level - 1))

    while x.shape[0] > 1024:
        half = x.shape[0] // 2
        b = x[half:]
        rb = jnp.concatenate(
            [_rev8(b[i:i + 1024]) for i in range(0, half, 1024)], axis=0)
        x = jnp.maximum(x[:half], rb)
        d = 8 << (_LOGK - 1)
        while d >= 8:
            x = _clean_stage(x, d)
            d //= 2

    iota = jax.lax.broadcasted_iota(jnp.int32, (1024, 1), 0)
    for j in range(3):
        dj = 1 << j
        bitb = (iota & dj) != 0
        partner = jnp.where(bitb, jnp.roll(x, dj, axis=0),
                            jnp.roll(x, -dj, axis=0))
        x = jnp.maximum(x, _rev8(partner))
        d = 8 << (_LOGK - 1)
        while d >= 8:
            x = _clean_stage(x, d)
            d //= 2

    cand = _rev8(x).reshape(_K, 8, cb)[:, 0, :]
    acc = o_ref[...]
    m = jnp.maximum(acc, cand)
    for u in range(_LOGK):
        m = _cmpx(m, 1 << (_LOGK - 1 - u), _LOGK)
    o_ref[...] = m


def kernel(inputs):
    b, h, w, c = inputs.shape
    out = pl.pallas_call(
        _kmax_kernel,
        grid=(b, c // _CB, h // _HB),
        in_specs=[
            pl.BlockSpec((None, _HB, None, _CB), lambda i, j, s: (i, s, 0, j)),
        ],
        out_specs=pl.BlockSpec((None, _K, None, _CB),
                               lambda i, j, s: (i, 0, 0, j)),
        out_shape=jax.ShapeDtypeStruct((b, _K, w, c), inputs.dtype),
    )(inputs)
    return out
